# baseline (speedup 1.0000x reference)
import jax
import jax.numpy as jnp
from jax.experimental import pallas as pl
from jax.experimental.pallas import tpu as pltpu

H, W, DH = 32, 32, 384
NCHUNK = 8
CH = H // NCHUNK


def _body(row_hbm, col_hbm, out_hbm, colv, rowv, buf, sems, insems):
    ccp = pltpu.make_async_copy(col_hbm.at[pl.ds(0, W)], colv, insems.at[0])
    ccp.start()
    rcp = pltpu.make_async_copy(row_hbm.at[pl.ds(0, H)], rowv, insems.at[1])
    rcp.start()
    ccp.wait()
    colb = jnp.broadcast_to(colv[...][None, :, :], (CH, W, DH))
    for k in range(NCHUNK):
        buf[CH * k:CH * (k + 1), :, 0:DH] = colb
    rcp.wait()
    copies = []
    for k in range(NCHUNK):
        row = rowv[CH * k:CH * (k + 1), :]
        buf[CH * k:CH * (k + 1), :, DH:2 * DH] = jnp.broadcast_to(
            row[:, None, :], (CH, W, DH))
        cp = pltpu.make_async_copy(
            buf.at[pl.ds(CH * k, CH)],
            out_hbm.at[pl.ds(CH * k, CH)],
            sems.at[k],
        )
        cp.start()
        copies.append(cp)
    for cp in copies:
        cp.wait()


def kernel(h, w, row_embed, col_embed):
    return pl.pallas_call(
        _body,
        in_specs=[
            pl.BlockSpec(memory_space=pl.ANY),
            pl.BlockSpec(memory_space=pl.ANY),
        ],
        out_specs=pl.BlockSpec(memory_space=pl.ANY),
        out_shape=jax.ShapeDtypeStruct((H, W, 2 * DH), jnp.float32),
        scratch_shapes=[
            pltpu.VMEM((W, DH), jnp.float32),
            pltpu.VMEM((H, DH), jnp.float32),
            pltpu.VMEM((H, W, 2 * DH), jnp.float32),
            pltpu.SemaphoreType.DMA((NCHUNK,)),
            pltpu.SemaphoreType.DMA((2,)),
        ],
    )(row_embed, col_embed)
